# baseline (speedup 1.0000x reference)
import functools

import jax
import jax.numpy as jnp
from jax import lax
from jax.experimental import pallas as pl
from jax.experimental.pallas import tpu as pltpu
from jax.experimental.pallas import tpu_sc as plsc

_TOL = 0.01
_MAXITER = 16

_N = 10000
_E = 320000
_CP = 48
_CHUNK = 128
_NBUF = 3
_NCHUNKS = 2592
_EPAD = _NCHUNKS * _CHUNK
_RPW = _N // 16
_NC = 1


def _proj_body(nf_ref, w_ref, b_ref, o_ref):
    o_ref[...] = (
        jnp.dot(nf_ref[...], w_ref[...], preferred_element_type=jnp.float32)
        + b_ref[...]
    )


def _project(nf, Wp, bp):
    N, D = nf.shape
    BN = 2000
    return pl.pallas_call(
        _proj_body,
        grid=(N // BN,),
        in_specs=[
            pl.BlockSpec((BN, D), lambda i: (i, 0)),
            pl.BlockSpec((D, _CP), lambda i: (0, 0)),
            pl.BlockSpec((1, _CP), lambda i: (0, 0)),
        ],
        out_specs=pl.BlockSpec((BN, _CP), lambda i: (i, 0)),
        out_shape=jax.ShapeDtypeStruct((N, _CP), jnp.float32),
    )(nf, Wp, bp.reshape(1, _CP))


def _make_sc_matvec(nc):
    cpw = _NCHUNKS // (nc * 16)

    @functools.partial(
        pl.kernel,
        out_type=jax.ShapeDtypeStruct((nc, _N, _CP), jnp.float32),
        mesh=plsc.VectorSubcoreMesh(
            core_axis_name="c", subcore_axis_name="s", num_cores=nc),
        compiler_params=pltpu.CompilerParams(use_tc_tiling_on_sc=False),
        scratch_types=(
            [
                pltpu.VMEM((cpw, _CHUNK), jnp.int32),
                pltpu.VMEM((cpw, _CHUNK), jnp.int32),
                pltpu.VMEM((cpw, _CHUNK), jnp.float32),
                pltpu.VMEM_SHARED((_N, _CP), jnp.float32),
            ]
            + [pltpu.VMEM((_CHUNK, _CP), jnp.float32)] * (2 * _NBUF)
            + [pltpu.SemaphoreType.DMA] * (2 * _NBUF)
        ),
    )
    def sc_matvec(vpad_hbm, col_hbm, row_hbm, adj_hbm, zeros_hbm, av_hbm,
                  col_all, row_all, adj_all, av_sh, *bufs_and_sems):
        _sc_matvec_body(cpw, col_all, row_all, adj_all, av_sh, bufs_and_sems,
                        vpad_hbm, col_hbm, row_hbm, adj_hbm, zeros_hbm, av_hbm)

    return sc_matvec


def _sc_matvec_body(cpw, col_all, row_all, adj_all, av_sh, bufs_and_sems,
                    vpad_hbm, col_hbm, row_hbm, adj_hbm, zeros_hbm, av_hbm):
    gb = bufs_and_sems[0:_NBUF]
    sb = bufs_and_sems[_NBUF:2 * _NBUF]
    gsem = bufs_and_sems[2 * _NBUF:3 * _NBUF]
    ssem = bufs_and_sems[3 * _NBUF:4 * _NBUF]
    cid = lax.axis_index("c")
    sid = lax.axis_index("s")
    base = (cid * 16 + sid) * cpw
    pltpu.sync_copy(col_hbm.at[pl.ds(base, cpw)], col_all)
    pltpu.sync_copy(row_hbm.at[pl.ds(base, cpw)], row_all)
    pltpu.sync_copy(adj_hbm.at[pl.ds(base, cpw)], adj_all)
    pltpu.sync_copy(zeros_hbm.at[pl.ds(sid * _RPW, _RPW)],
                    av_sh.at[pl.ds(sid * _RPW, _RPW)])
    plsc.subcore_barrier()

    for b in range(_NBUF):
        pltpu.async_copy(vpad_hbm.at[col_all.at[b]], gb[b], gsem[b])

    n_outer = cpw // _NBUF

    def outer_body(o, carry):
        for b in range(_NBUF):
            i = o * _NBUF + b
            pltpu.make_async_copy(vpad_hbm.at[col_all.at[b]], gb[b],
                                  gsem[b]).wait()
            @pl.when(o > 0)
            def _():
                pltpu.make_async_copy(sb[b], av_sh.at[row_all.at[b]],
                                      ssem[b]).wait()

            def edge_body(e16, c2):
                a16 = adj_all[i, pl.ds(e16 * 16, 16)]
                for j in range(16):
                    e = e16 * 16 + j
                    a = a16[j]
                    for k in range(3):
                        sb[b][e, pl.ds(16 * k, 16)] = (
                            gb[b][e, pl.ds(16 * k, 16)] * a
                        )
                return c2

            lax.fori_loop(0, _CHUNK // 16, edge_body, 0)

            @pl.when(o < n_outer - 1)
            def _():
                pltpu.async_copy(vpad_hbm.at[col_all.at[i + _NBUF]],
                                 gb[b], gsem[b])

            pltpu.async_copy(sb[b], av_sh.at[row_all.at[i]], ssem[b],
                             add=True)
        return carry

    lax.fori_loop(0, n_outer, outer_body, 0)
    for b in range(_NBUF):
        pltpu.make_async_copy(sb[b], av_sh.at[row_all.at[b]],
                              ssem[b]).wait()
    plsc.subcore_barrier()
    pltpu.sync_copy(av_sh.at[pl.ds(sid * _RPW, _RPW)],
                    av_hbm.at[cid, pl.ds(sid * _RPW, _RPW)])


_NIDP = 1024


@functools.partial(
    pl.kernel,
    out_type=jax.ShapeDtypeStruct((_NIDP, _CP), jnp.float32),
    mesh=plsc.VectorSubcoreMesh(core_axis_name="c", subcore_axis_name="s"),
    compiler_params=pltpu.CompilerParams(use_tc_tiling_on_sc=False),
    scratch_types=[
        pltpu.VMEM((32,), jnp.int32),
        pltpu.VMEM((32, _CP), jnp.float32),
        pltpu.SemaphoreType.DMA,
    ],
)
def _sc_ids_gather(sol_hbm, ids_hbm, out_hbm, ids_v, rows_v, sem):
    w = lax.axis_index("c") * 16 + lax.axis_index("s")
    pltpu.sync_copy(ids_hbm.at[w], ids_v)
    pltpu.async_copy(sol_hbm.at[ids_v], rows_v, sem).wait()
    pltpu.sync_copy(rows_v, out_hbm.at[pl.ds(w * 32, 32)])


def kernel(node_features, adj_values, e0, W, b, edge_index, ids):
    D, C = W.shape
    Wp = jnp.zeros((D, _CP), jnp.float32).at[:, :C].set(W)
    bp = jnp.zeros((_CP,), jnp.float32).at[:C].set(b)
    xpad = _project(node_features, Wp, bp)

    row = jnp.pad(edge_index[0], (0, _EPAD - _E)).reshape(_NCHUNKS, _CHUNK)
    col = jnp.pad(edge_index[1], (0, _EPAD - _E)).reshape(_NCHUNKS, _CHUNK)
    adj = jnp.pad(adj_values, (0, _EPAD - _E)).reshape(_NCHUNKS, _CHUNK)
    zeros = jnp.zeros((_N, _CP), jnp.float32)

    epsilon = jax.nn.sigmoid(e0)
    c = 1.0 - epsilon
    mv = _make_sc_matvec(_NC)

    def matvec(v):
        av2 = mv(v, col, row, adj, zeros)
        av = av2[0]
        for i in range(1, _NC):
            av = av + av2[i]
        return v - c * av

    sol, _ = jax.scipy.sparse.linalg.cg(matvec, xpad, tol=_TOL, maxiter=_MAXITER)

    ids_p = jnp.pad(ids, (0, _NIDP - ids.shape[0])).reshape(32, 32)
    outp = _sc_ids_gather(sol, ids_p)
    return outp[: ids.shape[0], :C]


# speedup vs baseline: 1.0684x; 1.0684x over previous
import functools

import jax
import jax.numpy as jnp
from jax import lax
from jax.experimental import pallas as pl
from jax.experimental.pallas import tpu as pltpu
from jax.experimental.pallas import tpu_sc as plsc

_TOL = 0.01
_MAXITER = 16

_N = 10000
_E = 320000
_CP = 48
_CHUNK = 128
_NBUF = 3
_NCHUNKS = 2592
_EPAD = _NCHUNKS * _CHUNK
_RPW = _N // 16
_NC = 2


def _proj_body(nf_ref, w_ref, b_ref, o_ref):
    o_ref[...] = (
        jnp.dot(nf_ref[...], w_ref[...], preferred_element_type=jnp.float32)
        + b_ref[...]
    )


def _project(nf, Wp, bp):
    N, D = nf.shape
    BN = 2000
    return pl.pallas_call(
        _proj_body,
        grid=(N // BN,),
        in_specs=[
            pl.BlockSpec((BN, D), lambda i: (i, 0)),
            pl.BlockSpec((D, _CP), lambda i: (0, 0)),
            pl.BlockSpec((1, _CP), lambda i: (0, 0)),
        ],
        out_specs=pl.BlockSpec((BN, _CP), lambda i: (i, 0)),
        out_shape=jax.ShapeDtypeStruct((N, _CP), jnp.float32),
    )(nf, Wp, bp.reshape(1, _CP))


def _make_sc_matvec(nc):
    cpw = _NCHUNKS // (nc * 16)

    @functools.partial(
        pl.kernel,
        out_type=jax.ShapeDtypeStruct((nc, _N, _CP), jnp.float32),
        mesh=plsc.VectorSubcoreMesh(
            core_axis_name="c", subcore_axis_name="s", num_cores=nc),
        compiler_params=pltpu.CompilerParams(use_tc_tiling_on_sc=False),
        scratch_types=(
            [
                pltpu.VMEM((cpw, _CHUNK), jnp.int32),
                pltpu.VMEM((cpw, _CHUNK), jnp.int32),
                pltpu.VMEM((cpw, _CHUNK), jnp.float32),
                pltpu.VMEM_SHARED((_N, _CP), jnp.float32),
            ]
            + [pltpu.VMEM((_CHUNK, _CP), jnp.float32)] * (2 * _NBUF)
            + [pltpu.SemaphoreType.DMA] * (2 * _NBUF)
        ),
    )
    def sc_matvec(vpad_hbm, col_hbm, row_hbm, adj_hbm, zeros_hbm, av_hbm,
                  col_all, row_all, adj_all, av_sh, *bufs_and_sems):
        _sc_matvec_body(cpw, col_all, row_all, adj_all, av_sh, bufs_and_sems,
                        vpad_hbm, col_hbm, row_hbm, adj_hbm, zeros_hbm, av_hbm)

    return sc_matvec


def _sc_matvec_body(cpw, col_all, row_all, adj_all, av_sh, bufs_and_sems,
                    vpad_hbm, col_hbm, row_hbm, adj_hbm, zeros_hbm, av_hbm):
    gb = bufs_and_sems[0:_NBUF]
    sb = bufs_and_sems[_NBUF:2 * _NBUF]
    gsem = bufs_and_sems[2 * _NBUF:3 * _NBUF]
    ssem = bufs_and_sems[3 * _NBUF:4 * _NBUF]
    cid = lax.axis_index("c")
    sid = lax.axis_index("s")
    base = (cid * 16 + sid) * cpw
    pltpu.sync_copy(col_hbm.at[pl.ds(base, cpw)], col_all)
    pltpu.sync_copy(row_hbm.at[pl.ds(base, cpw)], row_all)
    pltpu.sync_copy(adj_hbm.at[pl.ds(base, cpw)], adj_all)
    pltpu.sync_copy(zeros_hbm.at[pl.ds(sid * _RPW, _RPW)],
                    av_sh.at[pl.ds(sid * _RPW, _RPW)])
    plsc.subcore_barrier()

    for b in range(_NBUF):
        pltpu.async_copy(vpad_hbm.at[col_all.at[b]], gb[b], gsem[b])

    n_outer = cpw // _NBUF

    def outer_body(o, carry):
        for b in range(_NBUF):
            i = o * _NBUF + b
            pltpu.make_async_copy(vpad_hbm.at[col_all.at[b]], gb[b],
                                  gsem[b]).wait()
            @pl.when(o > 0)
            def _():
                pltpu.make_async_copy(sb[b], av_sh.at[row_all.at[b]],
                                      ssem[b]).wait()

            def edge_body(e16, c2):
                a16 = adj_all[i, pl.ds(e16 * 16, 16)]
                for j in range(16):
                    e = e16 * 16 + j
                    a = a16[j]
                    for k in range(3):
                        sb[b][e, pl.ds(16 * k, 16)] = (
                            gb[b][e, pl.ds(16 * k, 16)] * a
                        )
                return c2

            lax.fori_loop(0, _CHUNK // 16, edge_body, 0)

            @pl.when(o < n_outer - 1)
            def _():
                pltpu.async_copy(vpad_hbm.at[col_all.at[i + _NBUF]],
                                 gb[b], gsem[b])

            pltpu.async_copy(sb[b], av_sh.at[row_all.at[i]], ssem[b],
                             add=True)
        return carry

    lax.fori_loop(0, n_outer, outer_body, 0)
    for b in range(_NBUF):
        pltpu.make_async_copy(sb[b], av_sh.at[row_all.at[b]],
                              ssem[b]).wait()
    plsc.subcore_barrier()
    pltpu.sync_copy(av_sh.at[pl.ds(sid * _RPW, _RPW)],
                    av_hbm.at[cid, pl.ds(sid * _RPW, _RPW)])


_NIDP = 1024


@functools.partial(
    pl.kernel,
    out_type=jax.ShapeDtypeStruct((_NIDP, _CP), jnp.float32),
    mesh=plsc.VectorSubcoreMesh(core_axis_name="c", subcore_axis_name="s"),
    compiler_params=pltpu.CompilerParams(use_tc_tiling_on_sc=False),
    scratch_types=[
        pltpu.VMEM((32,), jnp.int32),
        pltpu.VMEM((32, _CP), jnp.float32),
        pltpu.SemaphoreType.DMA,
    ],
)
def _sc_ids_gather(sol_hbm, ids_hbm, out_hbm, ids_v, rows_v, sem):
    w = lax.axis_index("c") * 16 + lax.axis_index("s")
    pltpu.sync_copy(ids_hbm.at[w], ids_v)
    pltpu.async_copy(sol_hbm.at[ids_v], rows_v, sem).wait()
    pltpu.sync_copy(rows_v, out_hbm.at[pl.ds(w * 32, 32)])


def kernel(node_features, adj_values, e0, W, b, edge_index, ids):
    D, C = W.shape
    Wp = jnp.zeros((D, _CP), jnp.float32).at[:, :C].set(W)
    bp = jnp.zeros((_CP,), jnp.float32).at[:C].set(b)
    xpad = _project(node_features, Wp, bp)

    row = jnp.pad(edge_index[0], (0, _EPAD - _E)).reshape(_NCHUNKS, _CHUNK)
    col = jnp.pad(edge_index[1], (0, _EPAD - _E)).reshape(_NCHUNKS, _CHUNK)
    adj = jnp.pad(adj_values, (0, _EPAD - _E)).reshape(_NCHUNKS, _CHUNK)
    zeros = jnp.zeros((_N, _CP), jnp.float32)

    epsilon = jax.nn.sigmoid(e0)
    c = 1.0 - epsilon
    mv = _make_sc_matvec(_NC)

    def matvec(v):
        av2 = mv(v, col, row, adj, zeros)
        av = av2[0]
        for i in range(1, _NC):
            av = av + av2[i]
        return v - c * av

    sol, _ = jax.scipy.sparse.linalg.cg(matvec, xpad, tol=_TOL, maxiter=_MAXITER)

    ids_p = jnp.pad(ids, (0, _NIDP - ids.shape[0])).reshape(32, 32)
    outp = _sc_ids_gather(sol, ids_p)
    return outp[: ids.shape[0], :C]


# speedup vs baseline: 1.6404x; 1.5354x over previous
import functools

import jax
import jax.numpy as jnp
from jax import lax
from jax.experimental import pallas as pl
from jax.experimental.pallas import tpu as pltpu
from jax.experimental.pallas import tpu_sc as plsc

_TOL = 0.01
_MAXITER = 16

_N = 10000
_E = 320000
_CP = 48
_CHUNK = 128
_NBUF = 5
_NCHUNKS = 2560
_EPAD = _NCHUNKS * _CHUNK
_RPW = _N // 16
_NC = 2


def _proj_body(nf_ref, w_ref, b_ref, o_ref):
    o_ref[...] = (
        jnp.dot(nf_ref[...], w_ref[...], preferred_element_type=jnp.float32)
        + b_ref[...]
    )


def _project(nf, Wp, bp):
    N, D = nf.shape
    BN = 2000
    return pl.pallas_call(
        _proj_body,
        grid=(N // BN,),
        in_specs=[
            pl.BlockSpec((BN, D), lambda i: (i, 0)),
            pl.BlockSpec((D, _CP), lambda i: (0, 0)),
            pl.BlockSpec((1, _CP), lambda i: (0, 0)),
        ],
        out_specs=pl.BlockSpec((BN, _CP), lambda i: (i, 0)),
        out_shape=jax.ShapeDtypeStruct((N, _CP), jnp.float32),
    )(nf, Wp, bp.reshape(1, _CP))


def _make_sc_matvec(nc):
    cpw = _NCHUNKS // (nc * 16)

    @functools.partial(
        pl.kernel,
        out_type=jax.ShapeDtypeStruct((nc, _N, _CP), jnp.float32),
        mesh=plsc.VectorSubcoreMesh(
            core_axis_name="c", subcore_axis_name="s", num_cores=nc),
        compiler_params=pltpu.CompilerParams(use_tc_tiling_on_sc=False),
        scratch_types=(
            [
                pltpu.VMEM((cpw, _CHUNK), jnp.int32),
                pltpu.VMEM((cpw, _CHUNK), jnp.int32),
                pltpu.VMEM((cpw, _CHUNK), jnp.float32),
                pltpu.VMEM_SHARED((_N, _CP), jnp.float32),
            ]
            + [pltpu.VMEM((_CHUNK, _CP), jnp.float32)] * (2 * _NBUF)
            + [pltpu.SemaphoreType.DMA] * (2 * _NBUF)
        ),
    )
    def sc_matvec(vpad_hbm, col_hbm, row_hbm, adj_hbm, zeros_hbm, av_hbm,
                  col_all, row_all, adj_all, av_sh, *bufs_and_sems):
        _sc_matvec_body(cpw, col_all, row_all, adj_all, av_sh, bufs_and_sems,
                        vpad_hbm, col_hbm, row_hbm, adj_hbm, zeros_hbm, av_hbm)

    return sc_matvec


def _sc_matvec_body(cpw, col_all, row_all, adj_all, av_sh, bufs_and_sems,
                    vpad_hbm, col_hbm, row_hbm, adj_hbm, zeros_hbm, av_hbm):
    gb = bufs_and_sems[0:_NBUF]
    sb = bufs_and_sems[_NBUF:2 * _NBUF]
    gsem = bufs_and_sems[2 * _NBUF:3 * _NBUF]
    ssem = bufs_and_sems[3 * _NBUF:4 * _NBUF]
    cid = lax.axis_index("c")
    sid = lax.axis_index("s")
    base = (cid * 16 + sid) * cpw
    pltpu.sync_copy(col_hbm.at[pl.ds(base, cpw)], col_all)
    pltpu.sync_copy(row_hbm.at[pl.ds(base, cpw)], row_all)
    pltpu.sync_copy(adj_hbm.at[pl.ds(base, cpw)], adj_all)
    pltpu.sync_copy(zeros_hbm.at[pl.ds(sid * _RPW, _RPW)],
                    av_sh.at[pl.ds(sid * _RPW, _RPW)])
    plsc.subcore_barrier()

    for b in range(_NBUF):
        pltpu.async_copy(vpad_hbm.at[col_all.at[b]], gb[b], gsem[b])

    n_outer = cpw // _NBUF

    def outer_body(o, carry):
        for b in range(_NBUF):
            i = o * _NBUF + b
            pltpu.make_async_copy(vpad_hbm.at[col_all.at[b]], gb[b],
                                  gsem[b]).wait()
            @pl.when(o > 0)
            def _():
                pltpu.make_async_copy(sb[b], av_sh.at[row_all.at[b]],
                                      ssem[b]).wait()

            def edge_body(e16, c2):
                a16 = adj_all[i, pl.ds(e16 * 16, 16)]
                for j in range(16):
                    e = e16 * 16 + j
                    a = a16[j]
                    for k in range(3):
                        sb[b][e, pl.ds(16 * k, 16)] = (
                            gb[b][e, pl.ds(16 * k, 16)] * a
                        )
                return c2

            lax.fori_loop(0, _CHUNK // 16, edge_body, 0)

            @pl.when(o < n_outer - 1)
            def _():
                pltpu.async_copy(vpad_hbm.at[col_all.at[i + _NBUF]],
                                 gb[b], gsem[b])

            pltpu.async_copy(sb[b], av_sh.at[row_all.at[i]], ssem[b],
                             add=True)
        return carry

    lax.fori_loop(0, n_outer, outer_body, 0)
    for b in range(_NBUF):
        pltpu.make_async_copy(sb[b], av_sh.at[row_all.at[b]],
                              ssem[b]).wait()
    plsc.subcore_barrier()
    pltpu.sync_copy(av_sh.at[pl.ds(sid * _RPW, _RPW)],
                    av_hbm.at[cid, pl.ds(sid * _RPW, _RPW)])


_NIDP = 1024


@functools.partial(
    pl.kernel,
    out_type=jax.ShapeDtypeStruct((_NIDP, _CP), jnp.float32),
    mesh=plsc.VectorSubcoreMesh(core_axis_name="c", subcore_axis_name="s"),
    compiler_params=pltpu.CompilerParams(use_tc_tiling_on_sc=False),
    scratch_types=[
        pltpu.VMEM((32,), jnp.int32),
        pltpu.VMEM((32, _CP), jnp.float32),
        pltpu.SemaphoreType.DMA,
    ],
)
def _sc_ids_gather(sol_hbm, ids_hbm, out_hbm, ids_v, rows_v, sem):
    w = lax.axis_index("c") * 16 + lax.axis_index("s")
    pltpu.sync_copy(ids_hbm.at[w], ids_v)
    pltpu.async_copy(sol_hbm.at[ids_v], rows_v, sem).wait()
    pltpu.sync_copy(rows_v, out_hbm.at[pl.ds(w * 32, 32)])


def kernel(node_features, adj_values, e0, W, b, edge_index, ids):
    D, C = W.shape
    Wp = jnp.zeros((D, _CP), jnp.float32).at[:, :C].set(W)
    bp = jnp.zeros((_CP,), jnp.float32).at[:C].set(b)
    xpad = _project(node_features, Wp, bp)

    row = jnp.pad(edge_index[0], (0, _EPAD - _E)).reshape(_NCHUNKS, _CHUNK)
    col = jnp.pad(edge_index[1], (0, _EPAD - _E)).reshape(_NCHUNKS, _CHUNK)
    adj = jnp.pad(adj_values, (0, _EPAD - _E)).reshape(_NCHUNKS, _CHUNK)
    zeros = jnp.zeros((_N, _CP), jnp.float32)

    epsilon = jax.nn.sigmoid(e0)
    c = 1.0 - epsilon
    mv = _make_sc_matvec(_NC)

    def matvec(v):
        av2 = mv(v, col, row, adj, zeros)
        av = av2[0]
        for i in range(1, _NC):
            av = av + av2[i]
        return v - c * av

    sol, _ = jax.scipy.sparse.linalg.cg(matvec, xpad, tol=_TOL, maxiter=_MAXITER)

    ids_p = jnp.pad(ids, (0, _NIDP - ids.shape[0])).reshape(32, 32)
    outp = _sc_ids_gather(sol, ids_p)
    return outp[: ids.shape[0], :C]
